# Initial kernel scaffold; baseline (speedup 1.0000x reference)
import jax
import jax.numpy as jnp
from jax.experimental import pallas as pl


def kernel(x, edge_index, W1, b1, W2, b2):
    raise NotImplementedError("write your pallas kernel here")


# speedup vs baseline: 6.5452x; 6.5452x over previous
import functools

import jax
import jax.numpy as jnp
from jax import lax
from jax.experimental import pallas as pl
from jax.experimental.pallas import tpu as pltpu
from jax.experimental.pallas import tpu_sc as plsc

N, E, D = 10000, 160000, 256
DH = D // 2
NC, NS = 2, 16
L = 16
NPAD = 10240
EC = 128
NCHUNK = 1280
EP = (NCHUNK + NS) * EC
KT1 = NCHUNK // (NC * NS)
KT3 = NCHUNK // NS

mesh = plsc.VectorSubcoreMesh(core_axis_name="c", subcore_axis_name="s")


@functools.partial(
    pl.kernel,
    out_type=jax.ShapeDtypeStruct((NC, NPAD, 128), jnp.float32),
    mesh=mesh,
    scratch_types=[
        pltpu.VMEM((EC,), jnp.int32),
        pltpu.VMEM((EC, 128), jnp.float32),
        pltpu.VMEM((128, 128), jnp.float32),
        pltpu.VMEM_SHARED((NPAD, 128), jnp.float32),
        pltpu.SemaphoreType.DMA,
        pltpu.SemaphoreType.DMA,
    ],
)
def _deg_kernel(dst_hbm, out_hbm, idx_v, ones_v, buf_v, hist_s, isem, ssem):
    cidx = lax.axis_index("c")
    sidx = lax.axis_index("s")
    wid = cidx * NS + sidx

    def fill(i, _):
        ones_v[i // 8, pl.ds((i % 8) * L, L)] = jnp.full((L,), 1.0, jnp.float32)
        return 0
    lax.fori_loop(0, EC * 8, fill, 0)

    def zfill(i, _):
        buf_v[i // 8, pl.ds((i % 8) * L, L)] = jnp.zeros((L,), jnp.float32)
        return 0
    lax.fori_loop(0, 128 * 8, zfill, 0)
    for j in range(5):
        pltpu.sync_copy(buf_v, hist_s.at[pl.ds(sidx * 640 + j * 128, 128)])
    plsc.subcore_barrier()

    def body(k, _):
        c = wid + (NC * NS) * k
        pltpu.async_copy(dst_hbm.at[pl.ds(c * EC, EC)], idx_v, isem).wait()
        pltpu.async_copy(ones_v, hist_s.at[idx_v], ssem, add=True).wait()
        return 0
    lax.fori_loop(0, KT1, body, 0)
    plsc.subcore_barrier()

    for j in range(5):
        pltpu.sync_copy(hist_s.at[pl.ds(sidx * 640 + j * 128, 128)], buf_v)
        pltpu.sync_copy(buf_v, out_hbm.at[cidx, pl.ds(sidx * 640 + j * 128, 128)])


def _scale_body(hist_ref, x_ref, xs_ref, dinv_ref):
    deg = hist_ref[0, :, 0:1] + hist_ref[1, :, 0:1]
    di = jnp.where(deg > 0, lax.rsqrt(deg), 0.0)
    xs = x_ref[...] * di
    xs_ref[0] = xs[:, :DH]
    xs_ref[1] = xs[:, DH:]
    dinv_ref[...] = di


def _scale(hist, x):
    return pl.pallas_call(
        _scale_body,
        grid=(NPAD // 256,),
        in_specs=[
            pl.BlockSpec((NC, 256, 128), lambda b: (0, b, 0)),
            pl.BlockSpec((256, D), lambda b: (b, 0)),
        ],
        out_specs=[
            pl.BlockSpec((NC, 256, DH), lambda b: (0, b, 0)),
            pl.BlockSpec((256, 1), lambda b: (b, 0)),
        ],
        out_shape=[
            jax.ShapeDtypeStruct((NC, N, DH), jnp.float32),
            jax.ShapeDtypeStruct((NPAD, 1), jnp.float32),
        ],
    )(hist, x)


@functools.partial(
    pl.kernel,
    out_type=jax.ShapeDtypeStruct((NC, NPAD, DH), jnp.float32),
    mesh=mesh,
    scratch_types=[
        pltpu.VMEM((EC,), jnp.int32),
        pltpu.VMEM((EC,), jnp.int32),
        pltpu.VMEM((EC, DH), jnp.float32),
        pltpu.VMEM((EC,), jnp.int32),
        pltpu.VMEM((EC,), jnp.int32),
        pltpu.VMEM((EC, DH), jnp.float32),
        pltpu.VMEM_SHARED((NPAD, DH), jnp.float32),
        pltpu.SemaphoreType.DMA,
        pltpu.SemaphoreType.DMA,
        pltpu.SemaphoreType.DMA,
        pltpu.SemaphoreType.DMA,
        pltpu.SemaphoreType.DMA,
    ],
)
def _agg_kernel(src_hbm, dst_hbm, xs_hbm, out_hbm,
                idxsA, idxdA, rowsA, idxsB, idxdB, rowsB, acc_s,
                gsemA, gsemB, isem1, isem2, ssem):
    cidx = lax.axis_index("c")
    sidx = lax.axis_index("s")

    def zfill(i, _):
        rowsA[i // 8, pl.ds((i % 8) * L, L)] = jnp.zeros((L,), jnp.float32)
        return 0
    lax.fori_loop(0, EC * 8, zfill, 0)
    for j in range(5):
        pltpu.sync_copy(rowsA, acc_s.at[pl.ds(sidx * 640 + j * EC, EC)])
    plsc.subcore_barrier()

    pltpu.async_copy(src_hbm.at[pl.ds(sidx * EC, EC)], idxsA, isem1).wait()
    pltpu.async_copy(dst_hbm.at[pl.ds(sidx * EC, EC)], idxdA, isem2).wait()
    pltpu.async_copy(xs_hbm.at[cidx].at[idxsA], rowsA, gsemA)

    bufs = ((idxsA, idxdA, rowsA, gsemA), (idxsB, idxdB, rowsB, gsemB))

    def body(k2, _):
        for h in (0, 1):
            idxs_c, idxd_c, rows_c, gsem_c = bufs[h]
            idxs_n, idxd_n, rows_n, gsem_n = bufs[1 - h]
            cn = sidx + NS * (2 * k2 + h + 1)
            pltpu.async_copy(src_hbm.at[pl.ds(cn * EC, EC)], idxs_n, isem1).wait()
            pltpu.async_copy(dst_hbm.at[pl.ds(cn * EC, EC)], idxd_n, isem2).wait()
            pltpu.async_copy(xs_hbm.at[cidx].at[idxs_n], rows_n, gsem_n)
            pltpu.make_async_copy(
                xs_hbm.at[cidx].at[idxs_c], rows_c, gsem_c).wait()
            pltpu.async_copy(rows_c, acc_s.at[idxd_c], ssem, add=True).wait()
        return 0
    lax.fori_loop(0, KT3 // 2, body, 0)
    pltpu.make_async_copy(xs_hbm.at[cidx].at[idxsA], rowsA, gsemA).wait()
    plsc.subcore_barrier()

    for j in range(5):
        pltpu.sync_copy(acc_s.at[pl.ds(sidx * 640 + j * EC, EC)], rowsA)
        pltpu.sync_copy(rowsA, out_hbm.at[cidx, pl.ds(sidx * 640 + j * EC, EC)])


def _ffn_body(agg0_ref, agg1_ref, dinv_ref, w1_ref, b1_ref, w2_ref, b2_ref,
              out_ref):
    a0 = agg0_ref[0]
    a1 = agg1_ref[0]
    a = jnp.concatenate([a0, a1], axis=1) * dinv_ref[...]
    h = jnp.dot(a, w1_ref[...], preferred_element_type=jnp.float32) + b1_ref[...]
    h = 0.5 * h * (1.0 + lax.erf(h * jnp.float32(0.7071067811865476)))
    out_ref[...] = (
        jnp.dot(h, w2_ref[...], preferred_element_type=jnp.float32) + b2_ref[...]
    )


def _ffn(agg, dinv, W1, b1, W2, b2):
    return pl.pallas_call(
        _ffn_body,
        grid=(NPAD // 256,),
        in_specs=[
            pl.BlockSpec((1, 256, DH), lambda b: (0, b, 0)),
            pl.BlockSpec((1, 256, DH), lambda b: (1, b, 0)),
            pl.BlockSpec((256, 1), lambda b: (b, 0)),
            pl.BlockSpec((D, D), lambda b: (0, 0)),
            pl.BlockSpec((1, D), lambda b: (0, 0)),
            pl.BlockSpec((D, D), lambda b: (0, 0)),
            pl.BlockSpec((1, D), lambda b: (0, 0)),
        ],
        out_specs=pl.BlockSpec((256, D), lambda b: (b, 0)),
        out_shape=jax.ShapeDtypeStruct((NPAD, D), jnp.float32),
    )(agg, agg, dinv, W1, b1, W2, b2)


def kernel(x, edge_index, W1, b1, W2, b2):
    src = jnp.concatenate(
        [edge_index[0], jnp.zeros((EP - E,), jnp.int32)])
    dst = jnp.concatenate(
        [edge_index[1], jnp.full((EP - E,), NPAD - 1, jnp.int32)])
    hist = _deg_kernel(dst)
    xs, dinv = _scale(hist, x)
    agg = _agg_kernel(src, dst, xs)
    out = _ffn(agg, dinv, W1, b1.reshape(1, D), W2, b2.reshape(1, D))
    return out[:N]


# speedup vs baseline: 7.9579x; 1.2158x over previous
import functools

import jax
import jax.numpy as jnp
from jax import lax
from jax.experimental import pallas as pl
from jax.experimental.pallas import tpu as pltpu
from jax.experimental.pallas import tpu_sc as plsc

N, E, D = 10000, 160000, 256
DH = D // 2
NC, NS = 2, 16
L = 16
NPAD = 10240
EC = 128
NCHUNK = 1280
EP = (NCHUNK + NS) * EC
KT1 = NCHUNK // (NC * NS)
KT3 = NCHUNK // NS

mesh = plsc.VectorSubcoreMesh(core_axis_name="c", subcore_axis_name="s")


@functools.partial(
    pl.kernel,
    out_type=jax.ShapeDtypeStruct((NC, NPAD, 128), jnp.float32),
    mesh=mesh,
    scratch_types=[
        pltpu.VMEM((EC,), jnp.int32),
        pltpu.VMEM((EC,), jnp.int32),
        pltpu.VMEM((EC, 128), jnp.float32),
        pltpu.VMEM((128, 128), jnp.float32),
        pltpu.VMEM_SHARED((NPAD, 128), jnp.float32),
        pltpu.SemaphoreType.DMA,
        pltpu.SemaphoreType.DMA,
        pltpu.SemaphoreType.DMA,
    ],
)
def _deg_kernel(dst_hbm, out_hbm, idxA_v, idxB_v, ones_v, buf_v, hist_s,
                isem, ssemA, ssemB):
    cidx = lax.axis_index("c")
    sidx = lax.axis_index("s")
    wid = cidx * NS + sidx

    def fill(i, _):
        ones_v[i // 8, pl.ds((i % 8) * L, L)] = jnp.full((L,), 1.0, jnp.float32)
        return 0
    lax.fori_loop(0, EC * 8, fill, 0)

    def zfill(i, _):
        buf_v[i // 8, pl.ds((i % 8) * L, L)] = jnp.zeros((L,), jnp.float32)
        return 0
    lax.fori_loop(0, 128 * 8, zfill, 0)
    for j in range(5):
        pltpu.sync_copy(buf_v, hist_s.at[pl.ds(sidx * 640 + j * 128, 128)])
    plsc.subcore_barrier()

    c0 = wid * EC
    pltpu.async_copy(dst_hbm.at[pl.ds(c0, EC)], idxA_v, isem).wait()
    pltpu.async_copy(ones_v, hist_s.at[idxA_v], ssemA, add=True)

    def body(k, _):
        for h, (idx_c, ssem_c, idx_o, ssem_o) in enumerate(
            ((idxB_v, ssemB, idxA_v, ssemA), (idxA_v, ssemA, idxB_v, ssemB))):
            c = (wid + (NC * NS) * (2 * k + 1 + h)) * EC
            pltpu.async_copy(dst_hbm.at[pl.ds(c, EC)], idx_c, isem).wait()
            pltpu.async_copy(ones_v, hist_s.at[idx_c], ssem_c, add=True)
            pltpu.make_async_copy(ones_v, hist_s.at[idx_o], ssem_o).wait()
        return 0
    lax.fori_loop(0, (KT1 - 1) // 2, body, 0)
    cl = (wid + (NC * NS) * (KT1 - 1)) * EC
    pltpu.async_copy(dst_hbm.at[pl.ds(cl, EC)], idxB_v, isem).wait()
    pltpu.async_copy(ones_v, hist_s.at[idxB_v], ssemB, add=True)
    pltpu.make_async_copy(ones_v, hist_s.at[idxA_v], ssemA).wait()
    pltpu.make_async_copy(ones_v, hist_s.at[idxB_v], ssemB).wait()
    plsc.subcore_barrier()

    for j in range(5):
        pltpu.sync_copy(hist_s.at[pl.ds(sidx * 640 + j * 128, 128)], buf_v)
        pltpu.sync_copy(buf_v, out_hbm.at[cidx, pl.ds(sidx * 640 + j * 128, 128)])


def _scale_body(hist_ref, x_ref, xs_ref, dinv_ref):
    deg = hist_ref[0, :, 0:1] + hist_ref[1, :, 0:1]
    di = jnp.where(deg > 0, lax.rsqrt(deg), 0.0)
    xs = x_ref[...] * di
    xs_ref[0] = xs[:, :DH]
    xs_ref[1] = xs[:, DH:]
    dinv_ref[...] = di


def _scale(hist, x):
    return pl.pallas_call(
        _scale_body,
        grid=(NPAD // 256,),
        in_specs=[
            pl.BlockSpec((NC, 256, 128), lambda b: (0, b, 0)),
            pl.BlockSpec((256, D), lambda b: (b, 0)),
        ],
        out_specs=[
            pl.BlockSpec((NC, 256, DH), lambda b: (0, b, 0)),
            pl.BlockSpec((256, 1), lambda b: (b, 0)),
        ],
        out_shape=[
            jax.ShapeDtypeStruct((NC, N, DH), jnp.float32),
            jax.ShapeDtypeStruct((NPAD, 1), jnp.float32),
        ],
    )(hist, x)


@functools.partial(
    pl.kernel,
    out_type=jax.ShapeDtypeStruct((NC, NPAD, DH), jnp.float32),
    mesh=mesh,
    scratch_types=[
        pltpu.VMEM((EC,), jnp.int32),
        pltpu.VMEM((EC,), jnp.int32),
        pltpu.VMEM((EC, DH), jnp.float32),
        pltpu.VMEM((EC,), jnp.int32),
        pltpu.VMEM((EC,), jnp.int32),
        pltpu.VMEM((EC, DH), jnp.float32),
        pltpu.VMEM_SHARED((NPAD, DH), jnp.float32),
        pltpu.SemaphoreType.DMA,
        pltpu.SemaphoreType.DMA,
        pltpu.SemaphoreType.DMA,
        pltpu.SemaphoreType.DMA,
        pltpu.SemaphoreType.DMA,
        pltpu.SemaphoreType.DMA,
    ],
)
def _agg_kernel(src_hbm, dst_hbm, xs_hbm, out_hbm,
                idxsA, idxdA, rowsA, idxsB, idxdB, rowsB, acc_s,
                gsemA, gsemB, isem1, isem2, ssemA, ssemB):
    cidx = lax.axis_index("c")
    sidx = lax.axis_index("s")

    def zfill(i, _):
        rowsA[i // 8, pl.ds((i % 8) * L, L)] = jnp.zeros((L,), jnp.float32)
        return 0
    lax.fori_loop(0, EC * 8, zfill, 0)
    for j in range(5):
        pltpu.sync_copy(rowsA, acc_s.at[pl.ds(sidx * 640 + j * EC, EC)])
    plsc.subcore_barrier()

    bufA = (idxsA, idxdA, rowsA, gsemA, ssemA)
    bufB = (idxsB, idxdB, rowsB, gsemB, ssemB)

    def _load_idx(t, idxs_v, idxd_v):
        base = (sidx + NS * t) * EC
        pltpu.async_copy(src_hbm.at[pl.ds(base, EC)], idxs_v, isem1).wait()
        pltpu.async_copy(dst_hbm.at[pl.ds(base, EC)], idxd_v, isem2).wait()

    def _fire_gather(idxs_v, rows_v, gsem):
        pltpu.async_copy(xs_hbm.at[cidx].at[idxs_v], rows_v, gsem)

    def _wait_gather(idxs_v, rows_v, gsem):
        pltpu.make_async_copy(xs_hbm.at[cidx].at[idxs_v], rows_v, gsem).wait()

    def _fire_scatter(idxd_v, rows_v, ssem):
        pltpu.async_copy(rows_v, acc_s.at[idxd_v], ssem, add=True)

    def _wait_scatter(idxd_v, rows_v, ssem):
        pltpu.make_async_copy(rows_v, acc_s.at[idxd_v], ssem).wait()

    _load_idx(0, idxsA, idxdA)
    _fire_gather(idxsA, rowsA, gsemA)
    _load_idx(1, idxsB, idxdB)
    _fire_gather(idxsB, rowsB, gsemB)
    _wait_gather(idxsA, rowsA, gsemA)
    _fire_scatter(idxdA, rowsA, ssemA)

    def body(k2, _):
        for h, (cur, oth) in enumerate(((bufB, bufA), (bufA, bufB))):
            idxs_c, idxd_c, rows_c, gsem_c, ssem_c = cur
            idxs_o, idxd_o, rows_o, gsem_o, ssem_o = oth
            t = 2 * k2 + 1 + h
            pltpu.make_async_copy(rows_o, acc_s.at[idxd_o], ssem_o).wait()
            _load_idx(t + 1, idxs_o, idxd_o)
            _fire_gather(idxs_o, rows_o, gsem_o)
            _wait_gather(idxs_c, rows_c, gsem_c)
            _fire_scatter(idxd_c, rows_c, ssem_c)
        return 0
    lax.fori_loop(0, (KT3 - 2) // 2, body, 0)
    _wait_scatter(idxdA, rowsA, ssemA)
    _wait_gather(idxsB, rowsB, gsemB)
    _fire_scatter(idxdB, rowsB, ssemB)
    _wait_scatter(idxdB, rowsB, ssemB)
    plsc.subcore_barrier()

    for j in range(5):
        pltpu.sync_copy(acc_s.at[pl.ds(sidx * 640 + j * EC, EC)], rowsA)
        pltpu.sync_copy(rowsA, out_hbm.at[cidx, pl.ds(sidx * 640 + j * EC, EC)])


def _ffn_body(agg0_ref, agg1_ref, dinv_ref, w1_ref, b1_ref, w2_ref, b2_ref,
              out_ref):
    a0 = agg0_ref[0]
    a1 = agg1_ref[0]
    a = jnp.concatenate([a0, a1], axis=1) * dinv_ref[...]
    h = jnp.dot(a, w1_ref[...], preferred_element_type=jnp.float32) + b1_ref[...]
    h = 0.5 * h * (1.0 + lax.erf(h * jnp.float32(0.7071067811865476)))
    out_ref[...] = (
        jnp.dot(h, w2_ref[...], preferred_element_type=jnp.float32) + b2_ref[...]
    )


def _ffn(agg, dinv, W1, b1, W2, b2):
    return pl.pallas_call(
        _ffn_body,
        grid=(NPAD // 256,),
        in_specs=[
            pl.BlockSpec((1, 256, DH), lambda b: (0, b, 0)),
            pl.BlockSpec((1, 256, DH), lambda b: (1, b, 0)),
            pl.BlockSpec((256, 1), lambda b: (b, 0)),
            pl.BlockSpec((D, D), lambda b: (0, 0)),
            pl.BlockSpec((1, D), lambda b: (0, 0)),
            pl.BlockSpec((D, D), lambda b: (0, 0)),
            pl.BlockSpec((1, D), lambda b: (0, 0)),
        ],
        out_specs=pl.BlockSpec((256, D), lambda b: (b, 0)),
        out_shape=jax.ShapeDtypeStruct((NPAD, D), jnp.float32),
    )(agg, agg, dinv, W1, b1, W2, b2)


def kernel(x, edge_index, W1, b1, W2, b2):
    src = jnp.concatenate(
        [edge_index[0], jnp.zeros((EP - E,), jnp.int32)])
    dst = jnp.concatenate(
        [edge_index[1], jnp.full((EP - E,), NPAD - 1, jnp.int32)])
    hist = _deg_kernel(dst)
    xs, dinv = _scale(hist, x)
    agg = _agg_kernel(src, dst, xs)
    out = _ffn(agg, dinv, W1, b1.reshape(1, D), W2, b2.reshape(1, D))
    return out[:N]


# speedup vs baseline: 8.4696x; 1.0643x over previous
import functools

import jax
import jax.numpy as jnp
from jax import lax
from jax.experimental import pallas as pl
from jax.experimental.pallas import tpu as pltpu
from jax.experimental.pallas import tpu_sc as plsc

N, E, D = 10000, 160000, 256
DH = D // 2
NC, NS = 2, 16
L = 16
NPAD = 10240
EC = 128
NCHUNK = 1280
EP = (NCHUNK + NS) * EC
KT1 = NCHUNK // (NC * NS)
KT3 = NCHUNK // NS

mesh = plsc.VectorSubcoreMesh(core_axis_name="c", subcore_axis_name="s")


@functools.partial(
    pl.kernel,
    out_type=jax.ShapeDtypeStruct((NC, NPAD, 128), jnp.float32),
    mesh=mesh,
    scratch_types=[
        pltpu.VMEM((EC,), jnp.int32),
        pltpu.VMEM((EC,), jnp.int32),
        pltpu.VMEM((EC, 128), jnp.float32),
        pltpu.VMEM((128, 128), jnp.float32),
        pltpu.VMEM_SHARED((NPAD, 128), jnp.float32),
        pltpu.SemaphoreType.DMA,
        pltpu.SemaphoreType.DMA,
        pltpu.SemaphoreType.DMA,
    ],
)
def _deg_kernel(dst_hbm, out_hbm, idxA_v, idxB_v, ones_v, buf_v, hist_s,
                isem, ssemA, ssemB):
    cidx = lax.axis_index("c")
    sidx = lax.axis_index("s")
    wid = cidx * NS + sidx

    def fill(i, _):
        ones_v[i // 8, pl.ds((i % 8) * L, L)] = jnp.full((L,), 1.0, jnp.float32)
        return 0
    lax.fori_loop(0, EC * 8, fill, 0)

    def zfill(i, _):
        buf_v[i // 8, pl.ds((i % 8) * L, L)] = jnp.zeros((L,), jnp.float32)
        return 0
    lax.fori_loop(0, 128 * 8, zfill, 0)
    for j in range(5):
        pltpu.sync_copy(buf_v, hist_s.at[pl.ds(sidx * 640 + j * 128, 128)])
    plsc.subcore_barrier()

    c0 = wid * EC
    pltpu.async_copy(dst_hbm.at[pl.ds(c0, EC)], idxA_v, isem).wait()
    pltpu.async_copy(ones_v, hist_s.at[idxA_v], ssemA, add=True)

    def body(k, _):
        for h, (idx_c, ssem_c, idx_o, ssem_o) in enumerate(
            ((idxB_v, ssemB, idxA_v, ssemA), (idxA_v, ssemA, idxB_v, ssemB))):
            c = (wid + (NC * NS) * (2 * k + 1 + h)) * EC
            pltpu.async_copy(dst_hbm.at[pl.ds(c, EC)], idx_c, isem).wait()
            pltpu.async_copy(ones_v, hist_s.at[idx_c], ssem_c, add=True)
            pltpu.make_async_copy(ones_v, hist_s.at[idx_o], ssem_o).wait()
        return 0
    lax.fori_loop(0, (KT1 - 1) // 2, body, 0)
    cl = (wid + (NC * NS) * (KT1 - 1)) * EC
    pltpu.async_copy(dst_hbm.at[pl.ds(cl, EC)], idxB_v, isem).wait()
    pltpu.async_copy(ones_v, hist_s.at[idxB_v], ssemB, add=True)
    pltpu.make_async_copy(ones_v, hist_s.at[idxA_v], ssemA).wait()
    pltpu.make_async_copy(ones_v, hist_s.at[idxB_v], ssemB).wait()
    plsc.subcore_barrier()

    for j in range(5):
        pltpu.sync_copy(hist_s.at[pl.ds(sidx * 640 + j * 128, 128)], buf_v)
        pltpu.sync_copy(buf_v, out_hbm.at[cidx, pl.ds(sidx * 640 + j * 128, 128)])


def _scale_body(hist_ref, x_ref, xs_ref, dinv_ref):
    deg = hist_ref[0, :, 0:1] + hist_ref[1, :, 0:1]
    di = jnp.where(deg > 0, lax.rsqrt(deg), 0.0)
    xs = x_ref[...] * di
    xs_ref[0] = xs[:, :DH]
    xs_ref[1] = xs[:, DH:]
    dinv_ref[...] = di


def _scale(hist, x):
    return pl.pallas_call(
        _scale_body,
        grid=(NPAD // 256,),
        in_specs=[
            pl.BlockSpec((NC, 256, 128), lambda b: (0, b, 0)),
            pl.BlockSpec((256, D), lambda b: (b, 0)),
        ],
        out_specs=[
            pl.BlockSpec((NC, 256, DH), lambda b: (0, b, 0)),
            pl.BlockSpec((256, 1), lambda b: (b, 0)),
        ],
        out_shape=[
            jax.ShapeDtypeStruct((NC, N, DH), jnp.float32),
            jax.ShapeDtypeStruct((NPAD, 1), jnp.float32),
        ],
    )(hist, x)


@functools.partial(
    pl.kernel,
    out_type=jax.ShapeDtypeStruct((NC, NPAD, DH), jnp.float32),
    mesh=mesh,
    scratch_types=[
        pltpu.VMEM((2, EC), jnp.int32),
        pltpu.VMEM((EC, DH), jnp.float32),
        pltpu.VMEM((2, EC), jnp.int32),
        pltpu.VMEM((EC, DH), jnp.float32),
        pltpu.VMEM_SHARED((NPAD, DH), jnp.float32),
        pltpu.SemaphoreType.DMA,
        pltpu.SemaphoreType.DMA,
        pltpu.SemaphoreType.DMA,
        pltpu.SemaphoreType.DMA,
        pltpu.SemaphoreType.DMA,
    ],
)
def _agg_kernel(ed_hbm, xs_hbm, out_hbm,
                idxA, rowsA, idxB, rowsB, acc_s,
                gsemA, gsemB, isem1, ssemA, ssemB):
    cidx = lax.axis_index("c")
    sidx = lax.axis_index("s")

    def zfill(i, _):
        rowsA[i // 8, pl.ds((i % 8) * L, L)] = jnp.zeros((L,), jnp.float32)
        return 0
    lax.fori_loop(0, EC * 8, zfill, 0)
    for j in range(5):
        pltpu.sync_copy(rowsA, acc_s.at[pl.ds(sidx * 640 + j * EC, EC)])
    plsc.subcore_barrier()

    bufA = (idxA, rowsA, gsemA, ssemA)
    bufB = (idxB, rowsB, gsemB, ssemB)

    def _load_idx(t, idx_v):
        pltpu.async_copy(ed_hbm.at[sidx + NS * t], idx_v, isem1).wait()

    def _fire_gather(idx_v, rows_v, gsem):
        pltpu.async_copy(xs_hbm.at[cidx].at[idx_v.at[0]], rows_v, gsem)

    def _wait_gather(idx_v, rows_v, gsem):
        pltpu.make_async_copy(
            xs_hbm.at[cidx].at[idx_v.at[0]], rows_v, gsem).wait()

    def _fire_scatter(idx_v, rows_v, ssem):
        pltpu.async_copy(rows_v, acc_s.at[idx_v.at[1]], ssem, add=True)

    def _wait_scatter(idx_v, rows_v, ssem):
        pltpu.make_async_copy(rows_v, acc_s.at[idx_v.at[1]], ssem).wait()

    _load_idx(0, idxA)
    _fire_gather(idxA, rowsA, gsemA)
    _load_idx(1, idxB)
    _fire_gather(idxB, rowsB, gsemB)
    _wait_gather(idxA, rowsA, gsemA)
    _fire_scatter(idxA, rowsA, ssemA)

    def body(k2, _):
        for h, (cur, oth) in enumerate(((bufB, bufA), (bufA, bufB))):
            idx_c, rows_c, gsem_c, ssem_c = cur
            idx_o, rows_o, gsem_o, ssem_o = oth
            t = 2 * k2 + 1 + h
            pltpu.make_async_copy(rows_o, acc_s.at[idx_o.at[1]], ssem_o).wait()
            _load_idx(t + 1, idx_o)
            _fire_gather(idx_o, rows_o, gsem_o)
            _wait_gather(idx_c, rows_c, gsem_c)
            _fire_scatter(idx_c, rows_c, ssem_c)
        return 0
    lax.fori_loop(0, (KT3 - 2) // 2, body, 0)
    _wait_scatter(idxA, rowsA, ssemA)
    _wait_gather(idxB, rowsB, gsemB)
    _fire_scatter(idxB, rowsB, ssemB)
    _wait_scatter(idxB, rowsB, ssemB)
    plsc.subcore_barrier()

    for j in range(5):
        pltpu.sync_copy(acc_s.at[pl.ds(sidx * 640 + j * EC, EC)], rowsA)
        pltpu.sync_copy(rowsA, out_hbm.at[cidx, pl.ds(sidx * 640 + j * EC, EC)])


def _ffn_body(agg0_ref, agg1_ref, dinv_ref, w1_ref, b1_ref, w2_ref, b2_ref,
              out_ref):
    a0 = agg0_ref[0]
    a1 = agg1_ref[0]
    a = jnp.concatenate([a0, a1], axis=1) * dinv_ref[...]
    h = jnp.dot(a, w1_ref[...], preferred_element_type=jnp.float32) + b1_ref[...]
    h = 0.5 * h * (1.0 + lax.erf(h * jnp.float32(0.7071067811865476)))
    out_ref[...] = (
        jnp.dot(h, w2_ref[...], preferred_element_type=jnp.float32) + b2_ref[...]
    )


def _ffn(agg, dinv, W1, b1, W2, b2):
    return pl.pallas_call(
        _ffn_body,
        grid=(NPAD // 256,),
        in_specs=[
            pl.BlockSpec((1, 256, DH), lambda b: (0, b, 0)),
            pl.BlockSpec((1, 256, DH), lambda b: (1, b, 0)),
            pl.BlockSpec((256, 1), lambda b: (b, 0)),
            pl.BlockSpec((D, D), lambda b: (0, 0)),
            pl.BlockSpec((1, D), lambda b: (0, 0)),
            pl.BlockSpec((D, D), lambda b: (0, 0)),
            pl.BlockSpec((1, D), lambda b: (0, 0)),
        ],
        out_specs=pl.BlockSpec((256, D), lambda b: (b, 0)),
        out_shape=jax.ShapeDtypeStruct((NPAD, D), jnp.float32),
    )(agg, agg, dinv, W1, b1, W2, b2)


def kernel(x, edge_index, W1, b1, W2, b2):
    src = jnp.concatenate(
        [edge_index[0], jnp.zeros((EP - E,), jnp.int32)])
    dst = jnp.concatenate(
        [edge_index[1], jnp.full((EP - E,), NPAD - 1, jnp.int32)])
    hist = _deg_kernel(dst)
    xs, dinv = _scale(hist, x)
    ed = jnp.stack([src.reshape(-1, EC), dst.reshape(-1, EC)], axis=1)
    agg = _agg_kernel(ed, xs)
    out = _ffn(agg, dinv, W1, b1.reshape(1, D), W2, b2.reshape(1, D))
    return out[:N]
